# Initial kernel scaffold; baseline (speedup 1.0000x reference)
import jax
import jax.numpy as jnp
from jax.experimental import pallas as pl


def kernel(input, offsets, weight_h, weight_hash):
    raise NotImplementedError("write your pallas kernel here")


# speedup vs baseline: 2.3834x; 2.3834x over previous
import functools

import jax
import jax.numpy as jnp
from jax import lax
from jax.experimental import pallas as pl
from jax.experimental.pallas import tpu as pltpu
from jax.experimental.pallas import tpu_sc as plsc

HOT_NUMS = 50000
HASH_SIZE = 450000
EMBED_DIM = 64
BATCH = 16384

NC = 2
NS = 16
NW = NC * NS
BPW = BATCH // NW
CHUNK = 128
NCHUNK = BPW // CHUNK
LANES = 16


def _sc_body(inp_hbm, wh_hbm, whash_hbm, out_hbm,
             inp_v, idxh_v, idxc_v, mf_v, rows_h, rows_c, sem_h, sem_c):
    wid = lax.axis_index("s") * NC + lax.axis_index("c")
    base = wid * BPW

    pltpu.sync_copy(inp_hbm.at[pl.ds(base, BPW)], inp_v)

    def idx_body(j, carry):
        sl = pl.ds(j * LANES, LANES)
        v = inp_v[sl]
        a = jnp.abs(v)
        mf_v[sl] = jnp.where(lax.rem(v, 10) == 0, 1, 0)
        idxh_v[sl] = lax.rem(a, HOT_NUMS)
        idxc_v[sl] = lax.rem(a, HASH_SIZE)
        return carry

    lax.fori_loop(0, BPW // LANES, idx_body, 0)

    def gather_body(k, carry):
        rsl = pl.ds(k * CHUNK, CHUNK)
        pltpu.async_copy(wh_hbm.at[idxh_v.at[rsl]], rows_h.at[rsl], sem_h)
        pltpu.async_copy(whash_hbm.at[idxc_v.at[rsl]], rows_c.at[rsl], sem_c)
        return carry

    lax.fori_loop(0, NCHUNK, gather_body, 0)

    pltpu.make_async_copy(wh_hbm.at[pl.ds(0, BPW)], rows_h, sem_h).wait()
    pltpu.make_async_copy(whash_hbm.at[pl.ds(0, BPW)], rows_c, sem_c).wait()

    iota = lax.iota(jnp.int32, LANES)
    cols = [iota + LANES * k for k in range(EMBED_DIM // LANES)]
    zeros = jnp.zeros((LANES,), jnp.int32)

    def row_body(j, carry):
        jvec = zeros + j
        mb = plsc.load_gather(mf_v, [jvec]) != 0
        for k in range(EMBED_DIM // LANES):
            aval = rows_h[j, pl.ds(LANES * k, LANES)]
            plsc.store_scatter(rows_c, [jvec, cols[k]], aval, mask=mb)
        return carry

    lax.fori_loop(0, BPW, row_body, 0)

    pltpu.sync_copy(rows_c, out_hbm.at[pl.ds(base, BPW)])


_lookup = functools.partial(
    pl.kernel,
    out_type=jax.ShapeDtypeStruct((BATCH, EMBED_DIM), jnp.float32),
    mesh=plsc.VectorSubcoreMesh(core_axis_name="c", subcore_axis_name="s"),
    compiler_params=pltpu.CompilerParams(
        needs_layout_passes=False, use_tc_tiling_on_sc=False),
    scratch_types=[
        pltpu.VMEM((BPW,), jnp.int32),
        pltpu.VMEM((BPW,), jnp.int32),
        pltpu.VMEM((BPW,), jnp.int32),
        pltpu.VMEM((BPW,), jnp.int32),
        pltpu.VMEM((BPW, EMBED_DIM), jnp.float32),
        pltpu.VMEM((BPW, EMBED_DIM), jnp.float32),
        pltpu.SemaphoreType.DMA,
        pltpu.SemaphoreType.DMA,
    ],
)(_sc_body)


def kernel(input, offsets, weight_h, weight_hash):
    del offsets
    return _lookup(input, weight_h, weight_hash)


# speedup vs baseline: 3.8115x; 1.5992x over previous
import functools

import jax
import jax.numpy as jnp
from jax import lax
from jax.experimental import pallas as pl
from jax.experimental.pallas import tpu as pltpu
from jax.experimental.pallas import tpu_sc as plsc

HOT_NUMS = 50000
HASH_SIZE = 450000
EMBED_DIM = 64
BATCH = 16384

NC = 2
NS = 16
NW = NC * NS
BPW = BATCH // NW
CHUNK = 128
NCHUNK = BPW // CHUNK
LANES = 16


def _sc_body(inp_hbm, wh_hbm, whash_hbm, out_hbm,
             inp_v, idx_v, mf_v, buf0, buf1, sem0, sem1):
    wid = lax.axis_index("s") * NC + lax.axis_index("c")
    base = wid * BPW

    pltpu.sync_copy(inp_hbm.at[pl.ds(base, BPW)], inp_v)

    def idx_body(j, carry):
        sl = pl.ds(j * LANES, LANES)
        v = inp_v[sl]
        a = jnp.abs(v)
        hot = lax.rem(v, 10) == 0
        mf_v[sl] = jnp.where(hot, 1, 0)
        idx_v[sl] = jnp.where(hot, lax.rem(a, HOT_NUMS), lax.rem(a, HASH_SIZE))
        return carry

    lax.fori_loop(0, BPW // LANES, idx_body, 0)

    bufs = (buf0, buf1)
    sems = (sem0, sem1)

    def fill(c, buf, sem):
        def group_body(g, carry):
            sl = pl.ds(c * CHUNK + g * LANES, LANES)
            v = idx_v[sl]
            m = mf_v[sl]
            for k in range(LANES):
                s = v[k]
                r = g * LANES + k

                @pl.when(m[k] != 0)
                def _():
                    pltpu.async_copy(
                        wh_hbm.at[pl.ds(s, 1)], buf.at[pl.ds(r, 1)], sem)

                @pl.when(m[k] == 0)
                def _():
                    pltpu.async_copy(
                        whash_hbm.at[pl.ds(s, 1)], buf.at[pl.ds(r, 1)], sem)
            return carry

        lax.fori_loop(0, CHUNK // LANES, group_body, 0)

    def drain_and_flush(c, buf, sem):
        pltpu.make_async_copy(wh_hbm.at[pl.ds(0, CHUNK)], buf, sem).wait()
        pltpu.sync_copy(buf, out_hbm.at[pl.ds(base + c * CHUNK, CHUNK)])

    for c in range(NCHUNK):
        fill(c, bufs[c % 2], sems[c % 2])
        if c > 0:
            drain_and_flush(c - 1, bufs[(c - 1) % 2], sems[(c - 1) % 2])
    drain_and_flush(NCHUNK - 1, bufs[(NCHUNK - 1) % 2], sems[(NCHUNK - 1) % 2])


_lookup = functools.partial(
    pl.kernel,
    out_type=jax.ShapeDtypeStruct((BATCH, EMBED_DIM), jnp.float32),
    mesh=plsc.VectorSubcoreMesh(core_axis_name="c", subcore_axis_name="s"),
    compiler_params=pltpu.CompilerParams(needs_layout_passes=False),
    scratch_types=[
        pltpu.VMEM((BPW,), jnp.int32),
        pltpu.VMEM((BPW,), jnp.int32),
        pltpu.VMEM((BPW,), jnp.int32),
        pltpu.VMEM((CHUNK, EMBED_DIM), jnp.float32),
        pltpu.VMEM((CHUNK, EMBED_DIM), jnp.float32),
        pltpu.SemaphoreType.DMA,
        pltpu.SemaphoreType.DMA,
    ],
)(_sc_body)


def kernel(input, offsets, weight_h, weight_hash):
    del offsets
    return _lookup(input, weight_h, weight_hash)
